# Initial kernel scaffold; baseline (speedup 1.0000x reference)
import jax
import jax.numpy as jnp
from jax.experimental import pallas as pl


def kernel(relative_position_bias_table, absolute_position_bias, relative_position_index):
    raise NotImplementedError("write your pallas kernel here")


# speedup vs baseline: 9.3778x; 9.3778x over previous
import functools

import jax
import jax.numpy as jnp
from jax import lax
from jax.experimental import pallas as pl
from jax.experimental.pallas import tpu as pltpu
from jax.experimental.pallas import tpu_sc as plsc

T = 1235
NH = 16
NROWS = 3721
ROW_PAD = 1248
NCHUNK = ROW_PAD // 16
NC, NS = 2, 16
NW = NC * NS
RPW = (T + NW - 1) // NW
LV1, LV2 = 961, 1186


def _sc_body(table_hbm, bias_hbm, rpi_hbm, out_hbm, table_v, bias_v, pos_v, stage_v):
    wid = lax.axis_index("s") * NC + lax.axis_index("c")
    r0 = wid * RPW
    r1 = jnp.minimum(r0 + RPW, T)

    pltpu.sync_copy(table_hbm, table_v)
    pltpu.sync_copy(bias_hbm, bias_v)

    def row_body(r, carry):
        pltpu.sync_copy(rpi_hbm.at[r], pos_v)
        lv = (r >= LV1).astype(jnp.int32) + (r >= LV2).astype(jnp.int32)
        bias_vecs = [bias_v[lv, h, :] for h in range(NH)]

        def interp(off):
            pos = pos_v[pl.ds(off, 16)]
            pos = jnp.minimum(jnp.maximum(pos, 0.0), float(NROWS - 1))
            pf = pos.astype(jnp.int32)
            wc = pos - pf.astype(jnp.float32)
            wf = 1.0 - wc
            pc = jnp.minimum(pf + 1, NROWS - 1)
            return pf * NH, pc * NH, wf, wc

        def chunk_body(c, carry2):
            off = c * 16
            bf, bc, wf, wc = interp(off)
            for h in range(NH):
                vf = plsc.load_gather(table_v, [bf + h])
                vc = plsc.load_gather(table_v, [bc + h])
                stage_v[h, 0, pl.ds(off, 16)] = wf * vf + wc * vc + bias_vecs[h]
            return carry2

        lax.fori_loop(0, T // 16, chunk_body, 0, unroll=False)

        tail_off = (T // 16) * 16
        bf, bc, wf, wc = interp(tail_off)
        lane = lax.iota(jnp.int32, 16)
        tail_mask = lane < (T - tail_off)
        zero = jnp.zeros((16,), jnp.int32)
        for h in range(NH):
            vf = plsc.load_gather(table_v, [bf + h])
            vc = plsc.load_gather(table_v, [bc + h])
            res = wf * vf + wc * vc + bias_vecs[h]
            plsc.store_scatter(
                stage_v,
                [jnp.full((16,), h, jnp.int32), zero, tail_off + lane],
                res,
                mask=tail_mask,
            )

        pltpu.sync_copy(stage_v, out_hbm.at[:, pl.ds(r, 1), :])
        return carry

    lax.fori_loop(r0, r1, row_body, 0, unroll=False)


@jax.jit
def _run(table_flat, bias_flat, rpi_pad):
    mesh = plsc.VectorSubcoreMesh(core_axis_name="c", subcore_axis_name="s")
    return pl.kernel(
        _sc_body,
        mesh=mesh,
        out_type=jax.ShapeDtypeStruct((NH, T, T), jnp.float32),
        scratch_types=[
            pltpu.VMEM((NROWS * NH,), jnp.float32),
            pltpu.VMEM((3, NH, 16), jnp.float32),
            pltpu.VMEM((ROW_PAD,), jnp.float32),
            pltpu.VMEM((NH, 1, T), jnp.float32),
        ],
        compiler_params=pltpu.CompilerParams(needs_layout_passes=False),
    )(table_flat, bias_flat, rpi_pad)


def kernel(relative_position_bias_table, absolute_position_bias, relative_position_index):
    table_flat = relative_position_bias_table.reshape(-1)
    bias_splat = jnp.broadcast_to(
        absolute_position_bias.reshape(3, NH, 1), (3, NH, 16)
    )
    rpi_pad = jnp.pad(relative_position_index, ((0, 0), (0, ROW_PAD - T)))
    out = _run(table_flat, bias_splat, rpi_pad)
    return out.reshape(1, NH, 1, T, T)


# speedup vs baseline: 19.6632x; 2.0968x over previous
import jax
import jax.numpy as jnp
from jax import lax
from jax.experimental import pallas as pl
from jax.experimental.pallas import tpu as pltpu
from jax.experimental.pallas import tpu_sc as plsc

T = 1235
NH = 16
NROWS = 3721
ROW_PAD = 1248
NC, NS = 2, 16
NW = NC * NS
RPW = (T + NW - 1) // NW
LV1, LV2 = 961, 1186
HGRP = 8


def _sc_body(table_hbm, bias_hbm, rpi_hbm, out_hbm, table_v, bias_v, pos_v, stage_v, sem_in, sem_out):
    wid = lax.axis_index("s") * NC + lax.axis_index("c")
    r0 = wid * RPW
    r1 = jnp.minimum(r0 + RPW, T)
    nrows = r1 - r0

    pltpu.sync_copy(table_hbm, table_v)
    pltpu.sync_copy(bias_hbm, bias_v)
    pltpu.sync_copy(rpi_hbm.at[r0], pos_v.at[0])

    def row_body(idx, carry):
        r = r0 + idx
        slot = jnp.bitwise_and(idx, 1)

        @pl.when(idx + 1 < nrows)
        def _():
            pltpu.make_async_copy(
                rpi_hbm.at[r + 1], pos_v.at[1 - slot], sem_in
            ).start()

        @pl.when(idx > 0)
        def _():
            pltpu.make_async_copy(
                rpi_hbm.at[r], pos_v.at[slot], sem_in
            ).wait()

        @pl.when(idx >= 2)
        def _():
            pltpu.make_async_copy(
                stage_v.at[slot], out_hbm.at[:, pl.ds(r - 2, 1), :], sem_out
            ).wait()

        lv = (r >= LV1).astype(jnp.int32) + (r >= LV2).astype(jnp.int32)
        bias_vecs = [bias_v[lv, h, :] for h in range(NH)]

        def interp(off):
            pos = pos_v[slot, pl.ds(off, 16)]
            pos = jnp.minimum(jnp.maximum(pos, 0.0), float(NROWS - 1))
            pf = pos.astype(jnp.int32)
            wc = pos - pf.astype(jnp.float32)
            wf = 1.0 - wc
            pc = jnp.minimum(pf + 1, NROWS - 1)
            return pf * NH, pc * NH, wf, wc

        def head_group(g, bf, bc, wf, wc, emit):
            vfs = [plsc.load_gather(table_v, [bf + h]) for h in range(g, g + HGRP)]
            vcs = [plsc.load_gather(table_v, [bc + h]) for h in range(g, g + HGRP)]
            for k, h in enumerate(range(g, g + HGRP)):
                emit(h, wf * vfs[k] + wc * vcs[k] + bias_vecs[h])

        def chunk_body(c, carry2):
            off = c * 16
            bf, bc, wf, wc = interp(off)

            def emit(h, res):
                stage_v[slot, h, 0, pl.ds(off, 16)] = res

            for g in range(0, NH, HGRP):
                head_group(g, bf, bc, wf, wc, emit)
            return carry2

        lax.fori_loop(0, T // 16, chunk_body, 0, unroll=False)

        tail_off = (T // 16) * 16
        bf, bc, wf, wc = interp(tail_off)
        lane = lax.iota(jnp.int32, 16)
        tail_mask = lane < (T - tail_off)
        slot_vec = jnp.broadcast_to(slot, (16,))

        def emit_tail(h, res):
            plsc.store_scatter(
                stage_v,
                [slot_vec, jnp.full((16,), h, jnp.int32),
                 jnp.zeros((16,), jnp.int32), tail_off + lane],
                res,
                mask=tail_mask,
            )

        for g in range(0, NH, HGRP):
            head_group(g, bf, bc, wf, wc, emit_tail)

        pltpu.make_async_copy(
            stage_v.at[slot], out_hbm.at[:, pl.ds(r, 1), :], sem_out
        ).start()
        return carry

    lax.fori_loop(0, nrows, row_body, 0, unroll=False)

    @pl.when(nrows >= 2)
    def _():
        pltpu.make_async_copy(
            stage_v.at[0], out_hbm.at[:, pl.ds(r0, 1), :], sem_out
        ).wait()
    pltpu.make_async_copy(
        stage_v.at[0], out_hbm.at[:, pl.ds(r0, 1), :], sem_out
    ).wait()


@jax.jit
def _run(table_flat, bias_splat, rpi_pad):
    mesh = plsc.VectorSubcoreMesh(core_axis_name="c", subcore_axis_name="s")
    return pl.kernel(
        _sc_body,
        mesh=mesh,
        out_type=jax.ShapeDtypeStruct((NH, T, T), jnp.float32),
        scratch_types=[
            pltpu.VMEM((NROWS * NH,), jnp.float32),
            pltpu.VMEM((3, NH, 16), jnp.float32),
            pltpu.VMEM((2, ROW_PAD), jnp.float32),
            pltpu.VMEM((2, NH, 1, T), jnp.float32),
            pltpu.SemaphoreType.DMA,
            pltpu.SemaphoreType.DMA,
        ],
        compiler_params=pltpu.CompilerParams(needs_layout_passes=False),
    )(table_flat, bias_splat, rpi_pad)


def kernel(relative_position_bias_table, absolute_position_bias, relative_position_index):
    table_flat = relative_position_bias_table.reshape(-1)
    bias_splat = jnp.broadcast_to(
        absolute_position_bias.reshape(3, NH, 1), (3, NH, 16)
    )
    rpi_pad = jnp.pad(relative_position_index, ((0, 0), (0, ROW_PAD - T)))
    out = _run(table_flat, bias_splat, rpi_pad)
    return out.reshape(1, NH, 1, T, T)
